# baseline (speedup 1.0000x reference)
import jax
import jax.numpy as jnp
from jax.experimental import pallas as pl

_M_TILE = 856


def _head_body(xt_ref, xf_ref, wt_ref, wf_ref, b_ref, o_ref):
    dn = (((1,), (1,)), ((), ()))
    acc = jax.lax.dot_general(
        xt_ref[...], wt_ref[...], dn, preferred_element_type=jnp.float32
    )
    acc += jax.lax.dot_general(
        xf_ref[...], wf_ref[...], dn, preferred_element_type=jnp.float32
    )
    o_ref[...] = acc + b_ref[...]


def kernel(x_time, x_frequency, W, b):
    B, V, D, P = x_time.shape
    M = B * V
    K = D * P
    TW = W.shape[0]

    xt = x_time.reshape(M, K)
    xf = x_frequency.reshape(M, K)
    Wt = W[:, :K]
    Wf = W[:, K:]
    b2 = b.reshape(1, TW)

    grid = (M // _M_TILE,)
    out = pl.pallas_call(
        _head_body,
        grid=grid,
        in_specs=[
            pl.BlockSpec((_M_TILE, K), lambda i: (i, 0)),
            pl.BlockSpec((_M_TILE, K), lambda i: (i, 0)),
            pl.BlockSpec((TW, K), lambda i: (0, 0)),
            pl.BlockSpec((TW, K), lambda i: (0, 0)),
            pl.BlockSpec((1, TW), lambda i: (0, 0)),
        ],
        out_specs=pl.BlockSpec((_M_TILE, TW), lambda i: (i, 0)),
        out_shape=jax.ShapeDtypeStruct((M, TW), jnp.float32),
    )(xt, xf, Wt, Wf, b2)

    return out.reshape(B, V, TW)


# speedup vs baseline: 6.2016x; 6.2016x over previous
import jax
import jax.numpy as jnp
from jax.experimental import pallas as pl

_V_TILE = 107


def _head_body(xt_ref, xf_ref, wt_ref, wf_ref, b_ref, o_ref):
    p = pl.program_id(1)
    vt = xt_ref.shape[0]
    mb = vt * xt_ref.shape[2]
    d = xt_ref.shape[3]
    tw = o_ref.shape[1]

    xt = xt_ref[...].reshape(mb, d)
    xf = xf_ref[...].reshape(mb, d)
    wt = wt_ref[...].reshape(d, tw)
    wf = wf_ref[...].reshape(d, tw)
    dn = (((1,), (0,)), ((), ()))
    acc = jax.lax.dot_general(xt, wt, dn, preferred_element_type=jnp.float32)
    acc += jax.lax.dot_general(xf, wf, dn, preferred_element_type=jnp.float32)

    @pl.when(p == 0)
    def _init():
        o_ref[...] = acc + b_ref[...]

    @pl.when(p != 0)
    def _accum():
        o_ref[...] += acc


def kernel(x_time, x_frequency, W, b):
    B, V, D, P = x_time.shape
    K = D * P
    TW = W.shape[0]

    xt = jnp.transpose(x_time, (1, 3, 0, 2))
    xf = jnp.transpose(x_frequency, (1, 3, 0, 2))

    Wt = W[:, :K].reshape(TW, D, P).transpose(2, 1, 0)
    Wf = W[:, K:].reshape(TW, D, P).transpose(2, 1, 0)
    b2 = b.reshape(1, TW)

    grid = (V // _V_TILE, P)
    out = pl.pallas_call(
        _head_body,
        grid=grid,
        in_specs=[
            pl.BlockSpec((_V_TILE, 1, B, D), lambda i, p: (i, p, 0, 0)),
            pl.BlockSpec((_V_TILE, 1, B, D), lambda i, p: (i, p, 0, 0)),
            pl.BlockSpec((1, D, TW), lambda i, p: (p, 0, 0)),
            pl.BlockSpec((1, D, TW), lambda i, p: (p, 0, 0)),
            pl.BlockSpec((1, TW), lambda i, p: (0, 0)),
        ],
        out_specs=pl.BlockSpec((_V_TILE * B, TW), lambda i, p: (i, 0)),
        out_shape=jax.ShapeDtypeStruct((V * B, TW), jnp.float32),
    )(xt, xf, Wt, Wf, b2)

    return out.reshape(V, B, TW).transpose(1, 0, 2)
